# baseline (speedup 1.0000x reference)
import functools

import jax
import jax.numpy as jnp
from jax import lax
from jax.experimental import pallas as pl
from jax.experimental.pallas import tpu as pltpu
from jax.experimental.pallas import tpu_sc as plsc

_LANES = 16
_GBUF = 5
_PBUF = 2


def _grid(B, S):
    info = plsc.get_sparse_core_info()
    NC, NS = info.num_cores, info.num_subcores
    NW = NC * NS
    N = B * S
    assert N % NW == 0
    b_per_w = N // NW
    assert b_per_w % B == 0
    tp = b_per_w // B
    Rc = 16
    while tp % Rc != 0:
        Rc //= 2
    n_tc = tp // Rc
    return NC, NS, NW, b_per_w, tp, Rc, n_tc


@functools.lru_cache(maxsize=None)
def _build(B, S, V, D):
    NC, NS, NW, b_per_w, tp, Rc, n_tc = _grid(B, S)
    NU = B * n_tc
    assert NU >= _GBUF and n_tc >= _PBUF
    assert D % _LANES == 0

    mesh = plsc.VectorSubcoreMesh(core_axis_name="c", subcore_axis_name="s")

    scratch = (
        [pltpu.VMEM((B, n_tc, Rc), jnp.int32)]
        + [pltpu.VMEM((Rc, D), jnp.float32)] * (_GBUF + _PBUF)
        + [pltpu.SemaphoreType.DMA] * (2 * _GBUF + _PBUF + B)
    )

    @functools.partial(
        pl.kernel,
        out_type=jax.ShapeDtypeStruct((B * S, D), jnp.float32),
        mesh=mesh,
        scratch_types=scratch,
    )
    def k(x_hbm, tok_hbm, pe_hbm, out_hbm, idx_v, *rest):
        gbufs = rest[0:_GBUF]
        pbufs = rest[_GBUF:_GBUF + _PBUF]
        sems = rest[_GBUF + _PBUF:]
        gsems = sems[0:_GBUF]
        ssems = sems[_GBUF:2 * _GBUF]
        psems = sems[2 * _GBUF:2 * _GBUF + _PBUF]
        isems = sems[2 * _GBUF + _PBUF:2 * _GBUF + _PBUF + B]

        wid = lax.axis_index("s") * NC + lax.axis_index("c")
        t0 = wid * tp
        rows_per_b = S // Rc
        idx_dmas = []
        for bb in range(B):
            d = pltpu.make_async_copy(
                x_hbm.at[pl.ds(bb * rows_per_b + wid * n_tc, n_tc)],
                idx_v.at[bb], isems[bb])
            d.start()
            idx_dmas.append(d)

        pe_loads = {}
        loads = {}
        stores = {}

        def issue_pe(cc):
            pb = cc % _PBUF
            p = pltpu.make_async_copy(
                pe_hbm.at[pl.ds(t0 + cc * Rc, Rc)], pbufs[pb], psems[pb])
            p.start()
            pe_loads[cc] = p

        def issue_gather(u):
            gb = u % _GBUF
            cc, bb = divmod(u, B)
            g = pltpu.make_async_copy(
                tok_hbm.at[idx_v.at[bb, cc]], gbufs[gb], gsems[gb])
            g.start()
            loads[u] = g

        for cc in range(min(_PBUF, n_tc)):
            issue_pe(cc)
        n_head = min(_GBUF - 1, NU, B)
        for u in range(n_head):
            idx_dmas[u].wait()
            issue_gather(u)
        for d in idx_dmas[n_head:]:
            d.wait()
        for u in range(n_head, min(_GBUF - 1, NU)):
            issue_gather(u)

        for u in range(NU):
            cc, bb = divmod(u, B)
            gb = u % _GBUF
            pb = cc % _PBUF
            if bb == 0:
                pe_loads.pop(cc).wait()
            loads.pop(u).wait()
            nxt = u + _GBUF - 1
            if nxt < NU:
                if u >= 1:
                    stores.pop(u - 1).wait()
                issue_gather(nxt)

            def addj(j, carry, _gb=gb, _pb=pb):
                off = j * _LANES
                for r in range(Rc):
                    v = pbufs[_pb][r, pl.ds(off, _LANES)]
                    plsc.addupdate(
                        gbufs[_gb].at[r, pl.ds(off, _LANES)], v)
                return carry

            lax.fori_loop(0, D // _LANES, addj, 0, unroll=False)

            st = pltpu.make_async_copy(
                gbufs[gb],
                out_hbm.at[pl.ds(bb * S + t0 + cc * Rc, Rc)],
                ssems[gb])
            st.start()
            stores[u] = st

            if bb == B - 1 and cc + _PBUF < n_tc:
                issue_pe(cc + _PBUF)

        for u in sorted(stores):
            stores.pop(u).wait()

    return k


def kernel(x, tok_emb, pe):
    B, S = x.shape
    V, D = tok_emb.shape
    NC, NS, NW, b_per_w, tp, Rc, n_tc = _grid(B, S)
    x2 = x.astype(jnp.int32).reshape(B * S // Rc, Rc)
    pe_s = pe[:S, :]
    out = _build(B, S, V, D)(x2, tok_emb, pe_s)
    return out.reshape(B, S, D)


# speedup vs baseline: 1.1364x; 1.1364x over previous
import functools

import jax
import jax.numpy as jnp
from jax import lax
from jax.experimental import pallas as pl
from jax.experimental.pallas import tpu as pltpu
from jax.experimental.pallas import tpu_sc as plsc

_LANES = 16
_GBUF = 5
_PBUF = 2


def _grid(B, S):
    info = plsc.get_sparse_core_info()
    NC, NS = info.num_cores, info.num_subcores
    NW = NC * NS
    N = B * S
    assert N % NW == 0
    b_per_w = N // NW
    assert b_per_w % B == 0
    tp = b_per_w // B
    Rc = 16
    while tp % Rc != 0:
        Rc //= 2
    n_tc = tp // Rc
    return NC, NS, NW, b_per_w, tp, Rc, n_tc


@functools.lru_cache(maxsize=None)
def _build(B, S, V, D):
    NC, NS, NW, b_per_w, tp, Rc, n_tc = _grid(B, S)
    NU = B * n_tc
    assert NU >= _GBUF and n_tc >= _PBUF
    assert D % _LANES == 0

    mesh = plsc.VectorSubcoreMesh(core_axis_name="c", subcore_axis_name="s")

    scratch = (
        [pltpu.VMEM((B, n_tc, Rc), jnp.int32)]
        + [pltpu.VMEM((Rc, D), jnp.float32)] * (_GBUF + _PBUF)
        + [pltpu.SemaphoreType.DMA] * (2 * _GBUF + _PBUF + B)
    )

    @functools.partial(
        pl.kernel,
        out_type=jax.ShapeDtypeStruct((B * S, D), jnp.float32),
        mesh=mesh,
        scratch_types=scratch,
    )
    def k(x_hbm, tok_hbm, pe_hbm, out_hbm, idx_v, *rest):
        gbufs = rest[0:_GBUF]
        pbufs = rest[_GBUF:_GBUF + _PBUF]
        sems = rest[_GBUF + _PBUF:]
        gsems = sems[0:_GBUF]
        ssems = sems[_GBUF:2 * _GBUF]
        psems = sems[2 * _GBUF:2 * _GBUF + _PBUF]
        isems = sems[2 * _GBUF + _PBUF:2 * _GBUF + _PBUF + B]

        wid = lax.axis_index("s") * NC + lax.axis_index("c")
        t0 = wid * tp
        rows_per_b = S // Rc
        idx_dmas = []
        for bb in range(B):
            d = pltpu.make_async_copy(
                x_hbm.at[pl.ds(bb * rows_per_b + wid * n_tc, n_tc)],
                idx_v.at[bb], isems[bb])
            d.start()
            idx_dmas.append(d)

        pe_loads = {}
        loads = {}
        stores = {}

        def issue_pe(cc):
            pb = cc % _PBUF
            p = pltpu.make_async_copy(
                pe_hbm.at[pl.ds(t0 + cc * Rc, Rc)], pbufs[pb], psems[pb])
            p.start()
            pe_loads[cc] = p

        def issue_gather(u):
            gb = u % _GBUF
            cc, bb = divmod(u, B)
            g = pltpu.make_async_copy(
                tok_hbm.at[idx_v.at[bb, cc]], gbufs[gb], gsems[gb])
            g.start()
            loads[u] = g

        for cc in range(min(_PBUF, n_tc)):
            issue_pe(cc)
        n_head = min(_GBUF - 1, NU, B)
        for u in range(n_head):
            idx_dmas[u].wait()
            issue_gather(u)
        for d in idx_dmas[n_head:]:
            d.wait()
        for u in range(n_head, min(_GBUF - 1, NU)):
            issue_gather(u)

        for u in range(NU):
            cc, bb = divmod(u, B)
            gb = u % _GBUF
            pb = cc % _PBUF
            if bb == 0:
                pe_loads.pop(cc).wait()
            loads.pop(u).wait()

            def addj(j, carry, _gb=gb, _pb=pb):
                off = j * _LANES
                for r in range(Rc):
                    v = pbufs[_pb][r, pl.ds(off, _LANES)]
                    plsc.addupdate(
                        gbufs[_gb].at[r, pl.ds(off, _LANES)], v)
                return carry

            lax.fori_loop(0, D // _LANES, addj, 0, unroll=False)

            st = pltpu.make_async_copy(
                gbufs[gb],
                out_hbm.at[pl.ds(bb * S + t0 + cc * Rc, Rc)],
                ssems[gb])
            st.start()
            stores[u] = st

            if bb == B - 1 and cc + _PBUF < n_tc:
                issue_pe(cc + _PBUF)
            nxt = u + _GBUF - 1
            if nxt < NU:
                if u >= 1:
                    stores.pop(u - 1).wait()
                issue_gather(nxt)

        for u in sorted(stores):
            stores.pop(u).wait()

    return k


def kernel(x, tok_emb, pe):
    B, S = x.shape
    V, D = tok_emb.shape
    NC, NS, NW, b_per_w, tp, Rc, n_tc = _grid(B, S)
    x2 = x.astype(jnp.int32).reshape(B * S // Rc, Rc)
    pe_s = pe[:S, :]
    out = _build(B, S, V, D)(x2, tok_emb, pe_s)
    return out.reshape(B, S, D)


# speedup vs baseline: 1.1590x; 1.0199x over previous
import functools

import jax
import jax.numpy as jnp
from jax import lax
from jax.experimental import pallas as pl
from jax.experimental.pallas import tpu as pltpu
from jax.experimental.pallas import tpu_sc as plsc

_LANES = 16
_GBUF = 8
_PBUF = 2


def _grid(B, S):
    info = plsc.get_sparse_core_info()
    NC, NS = info.num_cores, info.num_subcores
    NW = NC * NS
    N = B * S
    assert N % NW == 0
    b_per_w = N // NW
    assert b_per_w % B == 0
    tp = b_per_w // B
    Rc = 8
    while tp % Rc != 0:
        Rc //= 2
    n_tc = tp // Rc
    return NC, NS, NW, b_per_w, tp, Rc, n_tc


@functools.lru_cache(maxsize=None)
def _build(B, S, V, D):
    NC, NS, NW, b_per_w, tp, Rc, n_tc = _grid(B, S)
    NU = B * n_tc
    assert NU >= _GBUF and n_tc >= _PBUF
    assert D % _LANES == 0

    mesh = plsc.VectorSubcoreMesh(core_axis_name="c", subcore_axis_name="s")

    scratch = (
        [pltpu.VMEM((B, n_tc, Rc), jnp.int32)]
        + [pltpu.VMEM((Rc, D), jnp.float32)] * (_GBUF + _PBUF)
        + [pltpu.SemaphoreType.DMA] * (2 * _GBUF + _PBUF + B)
    )

    @functools.partial(
        pl.kernel,
        out_type=jax.ShapeDtypeStruct((B * S, D), jnp.float32),
        mesh=mesh,
        scratch_types=scratch,
    )
    def k(x_hbm, tok_hbm, pe_hbm, out_hbm, idx_v, *rest):
        gbufs = rest[0:_GBUF]
        pbufs = rest[_GBUF:_GBUF + _PBUF]
        sems = rest[_GBUF + _PBUF:]
        gsems = sems[0:_GBUF]
        ssems = sems[_GBUF:2 * _GBUF]
        psems = sems[2 * _GBUF:2 * _GBUF + _PBUF]
        isems = sems[2 * _GBUF + _PBUF:2 * _GBUF + _PBUF + B]

        wid = lax.axis_index("s") * NC + lax.axis_index("c")
        t0 = wid * tp
        rows_per_b = S // Rc
        idx_dmas = []
        for bb in range(B):
            d = pltpu.make_async_copy(
                x_hbm.at[pl.ds(bb * rows_per_b + wid * n_tc, n_tc)],
                idx_v.at[bb], isems[bb])
            d.start()
            idx_dmas.append(d)

        pe_loads = {}
        loads = {}
        stores = {}

        def issue_pe(cc):
            pb = cc % _PBUF
            p = pltpu.make_async_copy(
                pe_hbm.at[pl.ds(t0 + cc * Rc, Rc)], pbufs[pb], psems[pb])
            p.start()
            pe_loads[cc] = p

        def issue_gather(u):
            gb = u % _GBUF
            cc, bb = divmod(u, B)
            g = pltpu.make_async_copy(
                tok_hbm.at[idx_v.at[bb, cc]], gbufs[gb], gsems[gb])
            g.start()
            loads[u] = g

        for cc in range(min(_PBUF, n_tc)):
            issue_pe(cc)
        n_head = min(_GBUF - 1, NU, B)
        for u in range(n_head):
            idx_dmas[u].wait()
            issue_gather(u)
        for d in idx_dmas[n_head:]:
            d.wait()
        for u in range(n_head, min(_GBUF - 1, NU)):
            issue_gather(u)

        for u in range(NU):
            cc, bb = divmod(u, B)
            gb = u % _GBUF
            pb = cc % _PBUF
            if bb == 0:
                pe_loads.pop(cc).wait()
            loads.pop(u).wait()

            def addj(j, carry, _gb=gb, _pb=pb):
                off = j * _LANES
                for r in range(Rc):
                    v = pbufs[_pb][r, pl.ds(off, _LANES)]
                    plsc.addupdate(
                        gbufs[_gb].at[r, pl.ds(off, _LANES)], v)
                return carry

            lax.fori_loop(0, D // _LANES, addj, 0, unroll=False)

            st = pltpu.make_async_copy(
                gbufs[gb],
                out_hbm.at[pl.ds(bb * S + t0 + cc * Rc, Rc)],
                ssems[gb])
            st.start()
            stores[u] = st

            if bb == B - 1 and cc + _PBUF < n_tc:
                issue_pe(cc + _PBUF)
            nxt = u + _GBUF - 1
            if nxt < NU:
                if u >= 1:
                    stores.pop(u - 1).wait()
                issue_gather(nxt)

        for u in sorted(stores):
            stores.pop(u).wait()

    return k


def kernel(x, tok_emb, pe):
    B, S = x.shape
    V, D = tok_emb.shape
    NC, NS, NW, b_per_w, tp, Rc, n_tc = _grid(B, S)
    x2 = x.astype(jnp.int32).reshape(B * S // Rc, Rc)
    pe_s = pe[:S, :]
    out = _build(B, S, V, D)(x2, tok_emb, pe_s)
    return out.reshape(B, S, D)


# speedup vs baseline: 1.1686x; 1.0083x over previous
import functools

import jax
import jax.numpy as jnp
from jax import lax
from jax.experimental import pallas as pl
from jax.experimental.pallas import tpu as pltpu
from jax.experimental.pallas import tpu_sc as plsc

_LANES = 16
_GBUF = 10
_PBUF = 2


def _grid(B, S):
    info = plsc.get_sparse_core_info()
    NC, NS = info.num_cores, info.num_subcores
    NW = NC * NS
    N = B * S
    assert N % NW == 0
    b_per_w = N // NW
    assert b_per_w % B == 0
    tp = b_per_w // B
    Rc = 8
    while tp % Rc != 0:
        Rc //= 2
    n_tc = tp // Rc
    return NC, NS, NW, b_per_w, tp, Rc, n_tc


@functools.lru_cache(maxsize=None)
def _build(B, S, V, D):
    NC, NS, NW, b_per_w, tp, Rc, n_tc = _grid(B, S)
    NU = B * n_tc
    assert NU >= _GBUF and n_tc >= _PBUF
    assert D % _LANES == 0

    mesh = plsc.VectorSubcoreMesh(core_axis_name="c", subcore_axis_name="s")

    scratch = (
        [pltpu.VMEM((B, n_tc, Rc), jnp.int32)]
        + [pltpu.VMEM((Rc, D), jnp.float32)] * (_GBUF + _PBUF)
        + [pltpu.SemaphoreType.DMA] * (2 * _GBUF + _PBUF + B)
    )

    @functools.partial(
        pl.kernel,
        out_type=jax.ShapeDtypeStruct((B * S, D), jnp.float32),
        mesh=mesh,
        scratch_types=scratch,
    )
    def k(x_hbm, tok_hbm, pe_hbm, out_hbm, idx_v, *rest):
        gbufs = rest[0:_GBUF]
        pbufs = rest[_GBUF:_GBUF + _PBUF]
        sems = rest[_GBUF + _PBUF:]
        gsems = sems[0:_GBUF]
        ssems = sems[_GBUF:2 * _GBUF]
        psems = sems[2 * _GBUF:2 * _GBUF + _PBUF]
        isems = sems[2 * _GBUF + _PBUF:2 * _GBUF + _PBUF + B]

        wid = lax.axis_index("s") * NC + lax.axis_index("c")
        t0 = wid * tp
        rows_per_b = S // Rc
        idx_dmas = []
        for bb in range(B):
            d = pltpu.make_async_copy(
                x_hbm.at[pl.ds(bb * rows_per_b + wid * n_tc, n_tc)],
                idx_v.at[bb], isems[bb])
            d.start()
            idx_dmas.append(d)

        pe_loads = {}
        loads = {}
        stores = {}

        def issue_pe(cc):
            pb = cc % _PBUF
            p = pltpu.make_async_copy(
                pe_hbm.at[pl.ds(t0 + cc * Rc, Rc)], pbufs[pb], psems[pb])
            p.start()
            pe_loads[cc] = p

        def issue_gather(u):
            gb = u % _GBUF
            cc, bb = divmod(u, B)
            g = pltpu.make_async_copy(
                tok_hbm.at[idx_v.at[bb, cc]], gbufs[gb], gsems[gb])
            g.start()
            loads[u] = g

        for cc in range(min(_PBUF, n_tc)):
            issue_pe(cc)
        n_head = min(_GBUF - 1, NU, B)
        for u in range(n_head):
            idx_dmas[u].wait()
            issue_gather(u)
        for d in idx_dmas[n_head:]:
            d.wait()
        for u in range(n_head, min(_GBUF - 1, NU)):
            issue_gather(u)

        for u in range(NU):
            cc, bb = divmod(u, B)
            gb = u % _GBUF
            pb = cc % _PBUF
            if bb == 0:
                pe_loads.pop(cc).wait()
            loads.pop(u).wait()

            def addj(j, carry, _gb=gb, _pb=pb):
                off = j * _LANES
                for r in range(Rc):
                    v = pbufs[_pb][r, pl.ds(off, _LANES)]
                    plsc.addupdate(
                        gbufs[_gb].at[r, pl.ds(off, _LANES)], v)
                return carry

            lax.fori_loop(0, D // _LANES, addj, 0, unroll=False)

            st = pltpu.make_async_copy(
                gbufs[gb],
                out_hbm.at[pl.ds(bb * S + t0 + cc * Rc, Rc)],
                ssems[gb])
            st.start()
            stores[u] = st

            if bb == B - 1 and cc + _PBUF < n_tc:
                issue_pe(cc + _PBUF)
            nxt = u + _GBUF - 1
            if nxt < NU:
                if u >= 1:
                    stores.pop(u - 1).wait()
                issue_gather(nxt)

        for u in sorted(stores):
            stores.pop(u).wait()

    return k


def kernel(x, tok_emb, pe):
    B, S = x.shape
    V, D = tok_emb.shape
    NC, NS, NW, b_per_w, tp, Rc, n_tc = _grid(B, S)
    x2 = x.astype(jnp.int32).reshape(B * S // Rc, Rc)
    pe_s = pe[:S, :]
    out = _build(B, S, V, D)(x2, tok_emb, pe_s)
    return out.reshape(B, S, D)
